# baseline (speedup 1.0000x reference)
import math
import jax
import jax.numpy as jnp
from jax.experimental import pallas as pl
from jax.experimental.pallas import tpu as pltpu


def _qjl_score_kernel(q_ref, k_ref, p_ref, out_ref, *, scale):
    q = q_ref[0, 0]
    k = k_ref[0, 0]
    p = p_ref[...]

    prec = jax.lax.Precision.DEFAULT
    sq = jax.lax.dot_general(q, p, (((1,), (0,)), ((), ())),
                             preferred_element_type=jnp.float32,
                             precision=prec)
    sk = jax.lax.dot_general(k, p, (((1,), (0,)), ((), ())),
                             preferred_element_type=jnp.float32,
                             precision=prec)
    sgn = jnp.sign(sk).astype(jnp.bfloat16)
    sqb = sq.astype(jnp.bfloat16)
    scores = jax.lax.dot_general(sgn, sqb, (((1,), (1,)), ((), ())),
                                 preferred_element_type=jnp.float32)
    norm = jnp.sqrt(jnp.sum(k * k, axis=1, keepdims=True))
    out_ref[0, 0] = scores * (norm * scale)


def kernel(query, key, proj_dir_score):
    B, HQ, QL, D = query.shape
    _, HK, KV, _ = key.shape
    S = proj_dir_score.shape[1]
    n_rep = HQ // HK
    scale = math.sqrt(math.pi / 2.0) / float(S)

    q4 = query.reshape(B, HK, n_rep, D)

    out = pl.pallas_call(
        lambda qr, kr, pr, orf: _qjl_score_kernel(qr, kr, pr, orf, scale=scale),
        grid=(B, HK),
        in_specs=[
            pl.BlockSpec((1, 1, n_rep, D), lambda b, h: (b, h, 0, 0)),
            pl.BlockSpec((1, 1, KV, D), lambda b, h: (b, h, 0, 0)),
            pl.BlockSpec((D, S), lambda b, h: (0, 0)),
        ],
        out_specs=pl.BlockSpec((1, 1, KV, n_rep), lambda b, h: (b, h, 0, 0)),
        out_shape=jax.ShapeDtypeStruct((B, HK, KV, n_rep), jnp.float32),
        compiler_params=pltpu.CompilerParams(
            dimension_semantics=("parallel", "parallel"),
        ),
    )(q4, key, proj_dir_score)

    return out.transpose(0, 1, 3, 2).reshape(B, HQ, KV, 1)


# speedup vs baseline: 1.5129x; 1.5129x over previous
import math
import jax
import jax.numpy as jnp
from jax.experimental import pallas as pl
from jax.experimental.pallas import tpu as pltpu


def _qjl_score_kernel(q_ref, k_ref, p_ref, out_ref, *, scale):
    q = q_ref[0, 0]
    p = p_ref[...]
    KV = k_ref.shape[2]
    half = KV // 2

    prec = jax.lax.Precision.DEFAULT
    k1 = k_ref[0, 0, :half]
    k2 = k_ref[0, 0, half:]
    kq1 = jnp.concatenate([k1, q], axis=0)
    skq1 = jax.lax.dot_general(kq1, p, (((1,), (0,)), ((), ())),
                               preferred_element_type=jnp.float32,
                               precision=prec)
    sk2 = jax.lax.dot_general(k2, p, (((1,), (0,)), ((), ())),
                              preferred_element_type=jnp.float32,
                              precision=prec)
    sq = skq1[half:]
    sqb = (sq * scale).astype(jnp.bfloat16)

    for kh, skh, col0 in ((k1, skq1[:half], 0), (k2, sk2, half)):
        n2 = jnp.sum(kh * kh, axis=1, keepdims=True)
        rn = n2 * jax.lax.rsqrt(n2)
        sk_bits = jax.lax.bitcast_convert_type(skh, jnp.uint32)
        rn_bits = jax.lax.bitcast_convert_type(rn, jnp.uint32)
        sgnn_bits = (sk_bits & jnp.uint32(0x80000000)) | rn_bits
        sgnn = jax.lax.bitcast_convert_type(sgnn_bits, jnp.float32).astype(jnp.bfloat16)
        out_ref[0, 0, :, col0:col0 + half] = jax.lax.dot_general(
            sqb, sgnn, (((1,), (1,)), ((), ())),
            preferred_element_type=jnp.float32)


def kernel(query, key, proj_dir_score):
    B, HQ, QL, D = query.shape
    _, HK, KV, _ = key.shape
    S = proj_dir_score.shape[1]
    n_rep = HQ // HK
    scale = math.sqrt(math.pi / 2.0) / float(S)

    q4 = query.reshape(B, HK, n_rep, D)

    out = pl.pallas_call(
        lambda qr, kr, pr, orf: _qjl_score_kernel(qr, kr, pr, orf, scale=scale),
        grid=(B, HK),
        in_specs=[
            pl.BlockSpec((1, 1, n_rep, D), lambda b, h: (b, h, 0, 0)),
            pl.BlockSpec((1, 1, KV, D), lambda b, h: (b, h, 0, 0)),
            pl.BlockSpec((D, S), lambda b, h: (0, 0)),
        ],
        out_specs=pl.BlockSpec((1, 1, n_rep, KV), lambda b, h: (b, h, 0, 0)),
        out_shape=jax.ShapeDtypeStruct((B, HK, n_rep, KV), jnp.float32),
        compiler_params=pltpu.CompilerParams(
            dimension_semantics=("parallel", "parallel"),
        ),
    )(q4, key, proj_dir_score)

    return out.reshape(B, HQ, KV, 1)


# speedup vs baseline: 1.5955x; 1.0546x over previous
import math
import jax
import jax.numpy as jnp
from jax.experimental import pallas as pl
from jax.experimental.pallas import tpu as pltpu


def _qjl_score_kernel(q_ref, k_ref, p_ref, out_ref, *, scale):
    q = q_ref[0, 0]
    p = p_ref[...]
    KV = k_ref.shape[2]
    half = KV // 2

    prec = jax.lax.Precision.DEFAULT
    quarter = KV // 4
    sq = jax.lax.dot_general(q, p, (((1,), (0,)), ((), ())),
                             preferred_element_type=jnp.float32,
                             precision=prec)
    sqb = (sq * scale).astype(jnp.bfloat16)

    chunks = []
    for c in range(0, 4):
        kc = k_ref[0, 0, c * quarter:(c + 1) * quarter]
        skc = jax.lax.dot_general(kc, p, (((1,), (0,)), ((), ())),
                                  preferred_element_type=jnp.float32,
                                  precision=prec)
        chunks.append((kc, skc, c * quarter))

    for kh, skh, col0 in chunks:
        n2 = jnp.sum(kh * kh, axis=1, keepdims=True)
        rn = n2 * jax.lax.rsqrt(n2)
        sk_bits = jax.lax.bitcast_convert_type(skh, jnp.uint32)
        rn_bits = jax.lax.bitcast_convert_type(rn, jnp.uint32)
        sgnn_bits = (sk_bits & jnp.uint32(0x80000000)) | rn_bits
        sgnn = jax.lax.bitcast_convert_type(sgnn_bits, jnp.float32).astype(jnp.bfloat16)
        out_ref[0, 0, :, col0:col0 + quarter] = jax.lax.dot_general(
            sqb, sgnn, (((1,), (1,)), ((), ())),
            preferred_element_type=jnp.float32)


def kernel(query, key, proj_dir_score):
    B, HQ, QL, D = query.shape
    _, HK, KV, _ = key.shape
    S = proj_dir_score.shape[1]
    n_rep = HQ // HK
    scale = math.sqrt(math.pi / 2.0) / float(S)

    q4 = query.reshape(B, HK, n_rep, D)

    out = pl.pallas_call(
        lambda qr, kr, pr, orf: _qjl_score_kernel(qr, kr, pr, orf, scale=scale),
        grid=(B, HK),
        in_specs=[
            pl.BlockSpec((1, 1, n_rep, D), lambda b, h: (b, h, 0, 0)),
            pl.BlockSpec((1, 1, KV, D), lambda b, h: (b, h, 0, 0)),
            pl.BlockSpec((D, S), lambda b, h: (0, 0)),
        ],
        out_specs=pl.BlockSpec((1, 1, n_rep, KV), lambda b, h: (b, h, 0, 0)),
        out_shape=jax.ShapeDtypeStruct((B, HK, n_rep, KV), jnp.float32),
        compiler_params=pltpu.CompilerParams(
            dimension_semantics=("parallel", "parallel"),
        ),
    )(q4, key, proj_dir_score)

    return out.reshape(B, HQ, KV, 1)


# speedup vs baseline: 1.7456x; 1.0941x over previous
import math
import jax
import jax.numpy as jnp
from jax.experimental import pallas as pl
from jax.experimental.pallas import tpu as pltpu

_CHUNKS = 4


def _qjl_score_kernel(q_ref, k_ref, p_ref, out_ref, *, scale):
    p = p_ref[...]
    KV = k_ref.shape[2]
    step = KV // _CHUNKS
    prec = jax.lax.Precision.DEFAULT

    for h in range(k_ref.shape[1]):
        q = q_ref[0, h]
        sq = jax.lax.dot_general(q, p, (((1,), (0,)), ((), ())),
                                 preferred_element_type=jnp.float32,
                                 precision=prec)
        sqb = (sq * scale).astype(jnp.bfloat16)

        for c in range(_CHUNKS):
            kc = k_ref[0, h, c * step:(c + 1) * step]
            skc = jax.lax.dot_general(kc, p, (((1,), (0,)), ((), ())),
                                      preferred_element_type=jnp.float32,
                                      precision=prec)
            n2 = jnp.sum(kc * kc, axis=1, keepdims=True)
            rn = n2 * jax.lax.rsqrt(n2)
            sk_bits = jax.lax.bitcast_convert_type(skc, jnp.uint32)
            rn_bits = jax.lax.bitcast_convert_type(rn, jnp.uint32)
            sgnn_bits = (sk_bits & jnp.uint32(0x80000000)) | rn_bits
            sgnn = jax.lax.bitcast_convert_type(
                sgnn_bits, jnp.float32).astype(jnp.bfloat16)
            out_ref[0, h, :, c * step:(c + 1) * step] = jax.lax.dot_general(
                sqb, sgnn, (((1,), (1,)), ((), ())),
                preferred_element_type=jnp.float32)


def kernel(query, key, proj_dir_score):
    B, HQ, QL, D = query.shape
    _, HK, KV, _ = key.shape
    S = proj_dir_score.shape[1]
    n_rep = HQ // HK
    scale = math.sqrt(math.pi / 2.0) / float(S)
    hpair = 2

    q4 = query.reshape(B, HK, n_rep, D)

    out = pl.pallas_call(
        lambda qr, kr, pr, orf: _qjl_score_kernel(qr, kr, pr, orf, scale=scale),
        grid=(B, HK // hpair),
        in_specs=[
            pl.BlockSpec((1, hpair, n_rep, D), lambda b, h: (b, h, 0, 0)),
            pl.BlockSpec((1, hpair, KV, D), lambda b, h: (b, h, 0, 0)),
            pl.BlockSpec((D, S), lambda b, h: (0, 0)),
        ],
        out_specs=pl.BlockSpec((1, hpair, n_rep, KV), lambda b, h: (b, h, 0, 0)),
        out_shape=jax.ShapeDtypeStruct((B, HK, n_rep, KV), jnp.float32),
        compiler_params=pltpu.CompilerParams(
            dimension_semantics=("parallel", "parallel"),
        ),
    )(q4, key, proj_dir_score)

    return out.reshape(B, HQ, KV, 1)


# speedup vs baseline: 1.7728x; 1.0156x over previous
import math
import jax
import jax.numpy as jnp
from jax.experimental import pallas as pl
from jax.experimental.pallas import tpu as pltpu

_CHUNKS = 4


def _qjl_score_kernel(q_ref, k_ref, p_ref, out_ref, *, scale):
    p = p_ref[...]
    KV = k_ref.shape[2]
    step = KV // _CHUNKS
    prec = jax.lax.Precision.DEFAULT

    for h in range(k_ref.shape[1]):
        q = q_ref[0, h]
        sq = jax.lax.dot_general(q, p, (((1,), (0,)), ((), ())),
                                 preferred_element_type=jnp.float32,
                                 precision=prec)
        sqb = (sq * scale).astype(jnp.bfloat16)

        for c in range(_CHUNKS):
            kc = k_ref[0, h, c * step:(c + 1) * step]
            skc = jax.lax.dot_general(kc, p, (((1,), (0,)), ((), ())),
                                      preferred_element_type=jnp.float32,
                                      precision=prec)
            n2 = jnp.sum(kc * kc, axis=1, keepdims=True)
            rn = n2 * jax.lax.rsqrt(n2)
            sk_bits = jax.lax.bitcast_convert_type(skc, jnp.uint32)
            rn_bits = jax.lax.bitcast_convert_type(rn, jnp.uint32)
            sgnn_bits = (sk_bits & jnp.uint32(0x80000000)) | rn_bits
            sgnn = jax.lax.bitcast_convert_type(
                sgnn_bits, jnp.float32).astype(jnp.bfloat16)
            out_ref[0, h, :, c * step:(c + 1) * step] = jax.lax.dot_general(
                sqb, sgnn, (((1,), (1,)), ((), ())),
                preferred_element_type=jnp.float32)


def kernel(query, key, proj_dir_score):
    B, HQ, QL, D = query.shape
    _, HK, KV, _ = key.shape
    S = proj_dir_score.shape[1]
    n_rep = HQ // HK
    scale = math.sqrt(math.pi / 2.0) / float(S)
    hpair = 4

    q4 = query.reshape(B, HK, n_rep, D)

    out = pl.pallas_call(
        lambda qr, kr, pr, orf: _qjl_score_kernel(qr, kr, pr, orf, scale=scale),
        grid=(B, HK // hpair),
        in_specs=[
            pl.BlockSpec((1, hpair, n_rep, D), lambda b, h: (b, h, 0, 0)),
            pl.BlockSpec((1, hpair, KV, D), lambda b, h: (b, h, 0, 0)),
            pl.BlockSpec((D, S), lambda b, h: (0, 0)),
        ],
        out_specs=pl.BlockSpec((1, hpair, n_rep, KV), lambda b, h: (b, h, 0, 0)),
        out_shape=jax.ShapeDtypeStruct((B, HK, n_rep, KV), jnp.float32),
        compiler_params=pltpu.CompilerParams(
            dimension_semantics=("parallel", "parallel"),
        ),
    )(q4, key, proj_dir_score)

    return out.reshape(B, HQ, KV, 1)


# speedup vs baseline: 1.9075x; 1.0759x over previous
import math
import jax
import jax.numpy as jnp
from jax.experimental import pallas as pl
from jax.experimental.pallas import tpu as pltpu

_CHUNKS = 2


def _qjl_score_kernel(q_ref, k_ref, p_ref, out_ref, *, scale):
    p = p_ref[...]
    KV = k_ref.shape[2]
    step = KV // _CHUNKS
    prec = jax.lax.Precision.DEFAULT

    HP = k_ref.shape[1]
    sqbs = []
    for h in range(HP):
        q = q_ref[0, h]
        sq = jax.lax.dot_general(q, p, (((1,), (0,)), ((), ())),
                                 preferred_element_type=jnp.float32,
                                 precision=prec)
        sqbs.append((sq * scale).astype(jnp.bfloat16))

    for c in range(_CHUNKS):
        for h in range(HP):
            kc = k_ref[0, h, c * step:(c + 1) * step]
            skc = jax.lax.dot_general(kc, p, (((1,), (0,)), ((), ())),
                                      preferred_element_type=jnp.float32,
                                      precision=prec)
            n2 = jnp.sum(kc * kc, axis=1, keepdims=True)
            rn = n2 * jax.lax.rsqrt(n2)
            sk_bits = jax.lax.bitcast_convert_type(skc, jnp.uint32)
            rn_bits = jax.lax.bitcast_convert_type(rn, jnp.uint32)
            sgnn_bits = (sk_bits & jnp.uint32(0x80000000)) | rn_bits
            sgnn = jax.lax.bitcast_convert_type(
                sgnn_bits, jnp.float32).astype(jnp.bfloat16)
            out_ref[0, h, :, c * step:(c + 1) * step] = jax.lax.dot_general(
                sqbs[h], sgnn, (((1,), (1,)), ((), ())),
                preferred_element_type=jnp.float32)


def kernel(query, key, proj_dir_score):
    B, HQ, QL, D = query.shape
    _, HK, KV, _ = key.shape
    S = proj_dir_score.shape[1]
    n_rep = HQ // HK
    scale = math.sqrt(math.pi / 2.0) / float(S)
    hpair = 4

    q4 = query.reshape(B, HK, n_rep, D)

    out = pl.pallas_call(
        lambda qr, kr, pr, orf: _qjl_score_kernel(qr, kr, pr, orf, scale=scale),
        grid=(B, HK // hpair),
        in_specs=[
            pl.BlockSpec((1, hpair, n_rep, D), lambda b, h: (b, h, 0, 0)),
            pl.BlockSpec((1, hpair, KV, D), lambda b, h: (b, h, 0, 0)),
            pl.BlockSpec((D, S), lambda b, h: (0, 0)),
        ],
        out_specs=pl.BlockSpec((1, hpair, n_rep, KV), lambda b, h: (b, h, 0, 0)),
        out_shape=jax.ShapeDtypeStruct((B, HK, n_rep, KV), jnp.float32),
        compiler_params=pltpu.CompilerParams(
            dimension_semantics=("parallel", "parallel"),
        ),
    )(q4, key, proj_dir_score)

    return out.reshape(B, HQ, KV, 1)


# speedup vs baseline: 2.1699x; 1.1376x over previous
import math
import jax
import jax.numpy as jnp
from jax.experimental import pallas as pl
from jax.experimental.pallas import tpu as pltpu

_CHUNKS = 8


def _qjl_score_kernel(q_ref, k_ref, p_ref, out_ref, *, scale):
    p = p_ref[...]
    KV = k_ref.shape[2]
    step = KV // _CHUNKS
    prec = jax.lax.Precision.DEFAULT

    HP = k_ref.shape[1]
    sqbs = []
    for h in range(HP):
        q = q_ref[0, h]
        sq = jax.lax.dot_general(q, p, (((1,), (0,)), ((), ())),
                                 preferred_element_type=jnp.float32,
                                 precision=prec)
        sqbs.append((sq * scale).astype(jnp.bfloat16))

    work = []
    for c in range(_CHUNKS):
        for h in range(HP):
            work.append((c, h))

    def _sketch(c, h):
        kc = k_ref[0, h, c * step:(c + 1) * step]
        skc = jax.lax.dot_general(kc, p, (((1,), (0,)), ((), ())),
                                  preferred_element_type=jnp.float32,
                                  precision=prec)
        return kc, skc

    def _emit(c, h, kc, skc):
        n2 = jnp.sum(kc * kc, axis=1, keepdims=True)
        rn = n2 * jax.lax.rsqrt(n2)
        sk_bits = jax.lax.bitcast_convert_type(skc, jnp.uint32)
        rn_bits = jax.lax.bitcast_convert_type(rn, jnp.uint32)
        sgnn_bits = (sk_bits & jnp.uint32(0x80000000)) | rn_bits
        sgnn = jax.lax.bitcast_convert_type(
            sgnn_bits, jnp.float32).astype(jnp.bfloat16)
        out_ref[0, h, :, c * step:(c + 1) * step] = jax.lax.dot_general(
            sqbs[h], sgnn, (((1,), (1,)), ((), ())),
            preferred_element_type=jnp.float32)

    from collections import deque
    pend = deque()
    for (c, h) in work:
        kc, skc = _sketch(c, h)
        pend.append((c, h, kc, skc))
        if len(pend) > 2:
            _emit(*pend.popleft())
    while pend:
        _emit(*pend.popleft())


def kernel(query, key, proj_dir_score):
    B, HQ, QL, D = query.shape
    _, HK, KV, _ = key.shape
    S = proj_dir_score.shape[1]
    n_rep = HQ // HK
    scale = math.sqrt(math.pi / 2.0) / float(S)
    hpair = 4

    q4 = query.reshape(B, HK, n_rep, D)

    out = pl.pallas_call(
        lambda qr, kr, pr, orf: _qjl_score_kernel(qr, kr, pr, orf, scale=scale),
        grid=(B, HK // hpair),
        in_specs=[
            pl.BlockSpec((1, hpair, n_rep, D), lambda b, h: (b, h, 0, 0)),
            pl.BlockSpec((1, hpair, KV, D), lambda b, h: (b, h, 0, 0)),
            pl.BlockSpec((D, S), lambda b, h: (0, 0)),
        ],
        out_specs=pl.BlockSpec((1, hpair, n_rep, KV), lambda b, h: (b, h, 0, 0)),
        out_shape=jax.ShapeDtypeStruct((B, HK, n_rep, KV), jnp.float32),
        compiler_params=pltpu.CompilerParams(
            dimension_semantics=("parallel", "parallel"),
        ),
    )(q4, key, proj_dir_score)

    return out.reshape(B, HQ, KV, 1)


# speedup vs baseline: 2.1836x; 1.0063x over previous
import math
import jax
import jax.numpy as jnp
from jax.experimental import pallas as pl
from jax.experimental.pallas import tpu as pltpu

_CHUNKS = 8


def _qjl_score_kernel(q_ref, k_ref, p_ref, out_ref, *, scale):
    p = p_ref[...]
    KV = k_ref.shape[2]
    step = KV // _CHUNKS
    prec = jax.lax.Precision.DEFAULT

    HP = k_ref.shape[1]
    sqbs = []
    for h in range(HP):
        q = q_ref[0, h]
        sq = jax.lax.dot_general(q, p, (((1,), (0,)), ((), ())),
                                 preferred_element_type=jnp.float32,
                                 precision=prec)
        sqbs.append((sq * scale).astype(jnp.bfloat16))

    work = []
    for c in range(_CHUNKS):
        for h in range(HP):
            work.append((c, h))

    def _sketch(c, h):
        kc = k_ref[0, h, c * step:(c + 1) * step]
        skc = jax.lax.dot_general(kc, p, (((1,), (0,)), ((), ())),
                                  preferred_element_type=jnp.float32,
                                  precision=prec)
        return kc, skc

    def _emit(c, h, kc, skc):
        n2 = jnp.sum(kc * kc, axis=1, keepdims=True)
        rn = n2 * jax.lax.rsqrt(n2)
        rnb = rn.astype(jnp.bfloat16)
        rnb_bits = jax.lax.bitcast_convert_type(rnb, jnp.uint16)
        skb = skc.astype(jnp.bfloat16)
        skb_bits = jax.lax.bitcast_convert_type(skb, jnp.uint16)
        sgnn_bits = (skb_bits & jnp.uint16(0x8000)) | rnb_bits
        sgnn = jax.lax.bitcast_convert_type(sgnn_bits, jnp.bfloat16)
        out_ref[0, h, :, c * step:(c + 1) * step] = jax.lax.dot_general(
            sqbs[h], sgnn, (((1,), (1,)), ((), ())),
            preferred_element_type=jnp.float32)

    from collections import deque
    pend = deque()
    for (c, h) in work:
        kc, skc = _sketch(c, h)
        pend.append((c, h, kc, skc))
        if len(pend) > 2:
            _emit(*pend.popleft())
    while pend:
        _emit(*pend.popleft())


def kernel(query, key, proj_dir_score):
    B, HQ, QL, D = query.shape
    _, HK, KV, _ = key.shape
    S = proj_dir_score.shape[1]
    n_rep = HQ // HK
    scale = math.sqrt(math.pi / 2.0) / float(S)
    hpair = 8

    q4 = query.reshape(B, HK, n_rep, D)

    out = pl.pallas_call(
        lambda qr, kr, pr, orf: _qjl_score_kernel(qr, kr, pr, orf, scale=scale),
        grid=(B, HK // hpair),
        in_specs=[
            pl.BlockSpec((1, hpair, n_rep, D), lambda b, h: (b, h, 0, 0)),
            pl.BlockSpec((1, hpair, KV, D), lambda b, h: (b, h, 0, 0)),
            pl.BlockSpec((D, S), lambda b, h: (0, 0)),
        ],
        out_specs=pl.BlockSpec((1, hpair, n_rep, KV), lambda b, h: (b, h, 0, 0)),
        out_shape=jax.ShapeDtypeStruct((B, HK, n_rep, KV), jnp.float32),
        compiler_params=pltpu.CompilerParams(
            dimension_semantics=("parallel", "parallel"),
        ),
    )(q4, key, proj_dir_score)

    return out.reshape(B, HQ, KV, 1)
